# baseline (speedup 1.0000x reference)
import functools

import jax
import jax.numpy as jnp
from jax import lax
from jax.experimental import pallas as pl
from jax.experimental.pallas import tpu as pltpu
from jax.experimental.pallas import tpu_sc as plsc

NUM_ROWS = 4096 * 200
VOCAB = 1000000
DIM = 32
LANES = 16

_INFO = plsc.get_sparse_core_info()
_NC = _INFO.num_cores
_NS = _INFO.num_subcores
NW = _NC * _NS

TWIN = 128
NFULL = VOCAB // TWIN
TAIL = VOCAB - NFULL * TWIN
WPW = (NFULL + NW - 1) // NW
TAIL_W = NFULL % NW


def _relayout_body(wt_hbm, out_hbm, in_v, out_v, in_t, out_t, isem, osem):
  wid = lax.axis_index("s") * _NC + lax.axis_index("c")
  nwin_mine = (NFULL - wid + NW - 1) // NW
  iota = lax.iota(jnp.int32, LANES)

  def win_start(t):
    return pl.multiple_of((wid + t * NW) * TWIN, TWIN)

  def fetch(t, b):
    return pltpu.make_async_copy(
        wt_hbm.at[:, pl.ds(win_start(t), TWIN)], in_v.at[b], isem.at[b])

  def flush(t, b):
    orows = TWIN * DIM // 128
    r0 = pl.multiple_of((wid + t * NW) * orows, orows)
    return pltpu.make_async_copy(
        out_v.at[b], out_hbm.at[pl.ds(r0, orows)], osem.at[b])

  def transpose_block(b):
    for r in range(TWIN):
      for d0 in range(0, DIM, LANES):
        v = plsc.load_gather(
            in_v.at[b], [iota + d0, jnp.full((LANES,), r, jnp.int32)])
        flat = r * DIM + d0
        out_v[b, flat // 128, pl.ds(flat % 128, LANES)] = v

  fetch(0, 0).start()

  def pair_body(g, carry):
    for s in range(2):
      t = 2 * g + s

      @pl.when(t < nwin_mine)
      def _():
        @pl.when(t + 1 < nwin_mine)
        def _():
          fetch(t + 1, 1 - s).start()
        fetch(t, s).wait()
        @pl.when(t >= 2)
        def _():
          flush(t - 2, s).wait()
        transpose_block(s)
        flush(t, s).start()
    return carry

  lax.fori_loop(0, (WPW + 1) // 2, pair_body, 0)
  flush(0, 0).wait()
  flush(0, 1).wait()

  @pl.when(wid == TAIL_W)
  def _():
    pltpu.sync_copy(wt_hbm.at[:, pl.ds(NFULL * TWIN, TAIL)], in_t)
    for r in range(TAIL):
      for d0 in range(0, DIM, LANES):
        v = plsc.load_gather(
            in_t, [iota + d0, jnp.full((LANES,), r, jnp.int32)])
        flat = r * DIM + d0
        out_t[flat // 128, pl.ds(flat % 128, LANES)] = v
    pltpu.sync_copy(
        out_t, out_hbm.at[pl.ds(NFULL * TWIN * DIM // 128, TAIL * DIM // 128)])


@jax.jit
def _relayout(wt):
  mesh = plsc.VectorSubcoreMesh(core_axis_name="c", subcore_axis_name="s")
  run = pl.kernel(
      _relayout_body,
      out_type=jax.ShapeDtypeStruct((VOCAB * DIM // 128, 128), jnp.float32),
      mesh=mesh,
      scratch_types=[
          pltpu.VMEM((2, DIM, TWIN), jnp.float32),
          pltpu.VMEM((2, TWIN * DIM // 128, 128), jnp.float32),
          pltpu.VMEM((DIM, TAIL), jnp.float32),
          pltpu.VMEM((TAIL * DIM // 128, 128), jnp.float32),
          pltpu.SemaphoreType.DMA((2,)),
          pltpu.SemaphoreType.DMA((2,)),
      ],
      compiler_params=pltpu.CompilerParams(needs_layout_passes=False),
  )
  return run(wt)


CHUNK = 512
NBUF = 5
CHUNKS_TOTAL = NUM_ROWS // CHUNK
CHUNKS_PER_W = CHUNKS_TOTAL // NW
GROUPS = CHUNKS_PER_W // NBUF


def _embed_body(idx_hbm, table_hbm, out_hbm, idx_v, rows_v, gsem, osem):
  wid = lax.axis_index("s") * _NC + lax.axis_index("c")
  row0 = wid * CHUNKS_PER_W

  pltpu.sync_copy(idx_hbm.at[pl.ds(row0, CHUNKS_PER_W)], idx_v)

  def gather(j, b):
    return pltpu.make_async_copy(table_hbm.at[idx_v.at[j]], rows_v.at[b],
                                 gsem.at[b])

  def outcopy(j, b):
    return pltpu.make_async_copy(
        rows_v.at[b], out_hbm.at[pl.ds((row0 + j) * CHUNK, CHUNK)],
        osem.at[b])

  for b in range(NBUF):
    gather(b, b).start()

  def body(g, carry):
    j0 = g * NBUF
    for b in range(NBUF):
      gather(j0 + b, b).wait()
      outcopy(j0 + b, b).start()
    for b in range(NBUF):
      outcopy(j0 + b, b).wait()
      gather(j0 + NBUF + b, b).start()
    return carry

  lax.fori_loop(0, GROUPS - 1, body, 0)

  j0 = (GROUPS - 1) * NBUF
  for b in range(NBUF):
    gather(j0 + b, b).wait()
    outcopy(j0 + b, b).start()
  for b in range(NBUF):
    outcopy(j0 + b, b).wait()


@jax.jit
def _embed(x_flat2d, w_lin):
  mesh = plsc.VectorSubcoreMesh(core_axis_name="c", subcore_axis_name="s")
  run = pl.kernel(
      _embed_body,
      out_type=jax.ShapeDtypeStruct((NUM_ROWS, DIM), jnp.float32),
      mesh=mesh,
      scratch_types=[
          pltpu.VMEM((CHUNKS_PER_W, CHUNK), jnp.int32),
          pltpu.VMEM((NBUF, CHUNK, DIM), jnp.float32),
          pltpu.SemaphoreType.DMA((NBUF,)),
          pltpu.SemaphoreType.DMA((NBUF,)),
      ],
      compiler_params=pltpu.CompilerParams(use_tc_tiling_on_sc=False),
  )
  return run(x_flat2d, w_lin)


def kernel(x, weight):
  w128 = _relayout(weight.T)
  w_lin = w128.reshape(VOCAB, DIM)
  x_flat2d = x.reshape(CHUNKS_TOTAL, CHUNK).astype(jnp.int32)
  out = _embed(x_flat2d, w_lin)
  return out.reshape(x.shape + (DIM,))


# speedup vs baseline: 1.6011x; 1.6011x over previous
import functools

import jax
import jax.numpy as jnp
from jax import lax
from jax.experimental import pallas as pl
from jax.experimental.pallas import tpu as pltpu
from jax.experimental.pallas import tpu_sc as plsc

NUM_ROWS = 4096 * 200
VOCAB = 1000000
DIM = 32
LANES = 16

_INFO = plsc.get_sparse_core_info()
_NC = _INFO.num_cores
_NS = _INFO.num_subcores
NW = _NC * _NS

VB = 7936
ORPB = VB * DIM // 128


def _tc_relayout_body(wt_ref, out_ref, b_ref):
  b_ref[...] = jnp.transpose(wt_ref[...], (1, 0))
  for j in range(4):
    out_ref[:, DIM * j:DIM * (j + 1)] = b_ref[pl.Slice(j, ORPB, 4), :]


@jax.jit
def _relayout(wt):
  return pl.pallas_call(
      _tc_relayout_body,
      grid=(pl.cdiv(VOCAB, VB),),
      in_specs=[pl.BlockSpec((DIM, VB), lambda i: (0, i))],
      out_specs=pl.BlockSpec((ORPB, 128), lambda i: (i, 0)),
      out_shape=jax.ShapeDtypeStruct((VOCAB * DIM // 128, 128), jnp.float32),
      scratch_shapes=[pltpu.VMEM((VB, DIM), jnp.float32)],
  )(wt)


CHUNK = 512
NBUF = 5
CHUNKS_TOTAL = NUM_ROWS // CHUNK
CHUNKS_PER_W = CHUNKS_TOTAL // NW
GROUPS = CHUNKS_PER_W // NBUF


def _embed_body(idx_hbm, table_hbm, out_hbm, idx_v, rows_v, gsem, osem):
  wid = lax.axis_index("s") * _NC + lax.axis_index("c")
  row0 = wid * CHUNKS_PER_W

  pltpu.sync_copy(idx_hbm.at[pl.ds(row0, CHUNKS_PER_W)], idx_v)

  def gather(j, b):
    return pltpu.make_async_copy(table_hbm.at[idx_v.at[j]], rows_v.at[b],
                                 gsem.at[b])

  def outcopy(j, b):
    return pltpu.make_async_copy(
        rows_v.at[b], out_hbm.at[pl.ds((row0 + j) * CHUNK, CHUNK)],
        osem.at[b])

  for b in range(NBUF):
    gather(b, b).start()

  def body(g, carry):
    j0 = g * NBUF
    for b in range(NBUF):
      gather(j0 + b, b).wait()
      outcopy(j0 + b, b).start()
    for b in range(NBUF):
      outcopy(j0 + b, b).wait()
      gather(j0 + NBUF + b, b).start()
    return carry

  lax.fori_loop(0, GROUPS - 1, body, 0)

  j0 = (GROUPS - 1) * NBUF
  for b in range(NBUF):
    gather(j0 + b, b).wait()
    outcopy(j0 + b, b).start()
  for b in range(NBUF):
    outcopy(j0 + b, b).wait()


@jax.jit
def _embed(x_flat2d, w_lin):
  mesh = plsc.VectorSubcoreMesh(core_axis_name="c", subcore_axis_name="s")
  run = pl.kernel(
      _embed_body,
      out_type=jax.ShapeDtypeStruct((NUM_ROWS, DIM), jnp.float32),
      mesh=mesh,
      scratch_types=[
          pltpu.VMEM((CHUNKS_PER_W, CHUNK), jnp.int32),
          pltpu.VMEM((NBUF, CHUNK, DIM), jnp.float32),
          pltpu.SemaphoreType.DMA((NBUF,)),
          pltpu.SemaphoreType.DMA((NBUF,)),
      ],
      compiler_params=pltpu.CompilerParams(use_tc_tiling_on_sc=False),
  )
  return run(x_flat2d, w_lin)


def kernel(x, weight):
  w128 = _relayout(weight.T)
  w_lin = w128.reshape(VOCAB, DIM)
  x_flat2d = x.reshape(CHUNKS_TOTAL, CHUNK).astype(jnp.int32)
  out = _embed(x_flat2d, w_lin)
  return out.reshape(x.shape + (DIM,))


# speedup vs baseline: 2.0670x; 1.2910x over previous
import functools

import jax
import jax.numpy as jnp
from jax import lax
from jax.experimental import pallas as pl
from jax.experimental.pallas import tpu as pltpu
from jax.experimental.pallas import tpu_sc as plsc

NUM_ROWS = 4096 * 200
VOCAB = 1000000
DIM = 32
LANES = 16

_INFO = plsc.get_sparse_core_info()
_NC = _INFO.num_cores
_NS = _INFO.num_subcores
NW = _NC * _NS

VB = 7936
ORPB = VB * DIM // 128


NBLK = 126
VMAIN = NBLK * VB
VTAIL = VOCAB - VMAIN
TROWS = VTAIL * DIM // 128
OROWS = VOCAB * DIM // 128


def _tc_relayout_body(wt_ref, out_ref, b_ref):
  b_ref[...] = jnp.transpose(wt_ref[...], (1, 0))
  for j in range(4):
    out_ref[:, DIM * j:DIM * (j + 1)] = b_ref[pl.Slice(j, ORPB, 4), :]


def _tc_tail_body(wt_ref, _, out_ref, b_ref):
  b_ref[...] = jnp.transpose(wt_ref[...], (1, 0))
  for j in range(4):
    out_ref[:, DIM * j:DIM * (j + 1)] = b_ref[pl.Slice(j, TROWS, 4), :]


@jax.jit
def _relayout(wt):
  main = pl.pallas_call(
      _tc_relayout_body,
      grid=(NBLK,),
      in_specs=[pl.BlockSpec((DIM, VB), lambda i: (0, i))],
      out_specs=pl.BlockSpec((ORPB, 128), lambda i: (i, 0)),
      out_shape=jax.ShapeDtypeStruct((OROWS, 128), jnp.float32),
      scratch_shapes=[pltpu.VMEM((VB, DIM), jnp.float32)],
  )(wt)
  return pl.pallas_call(
      _tc_tail_body,
      grid=(1,),
      in_specs=[
          pl.BlockSpec((DIM, VTAIL), lambda i: (0, 0)),
          pl.BlockSpec(memory_space=pl.ANY),
      ],
      out_specs=pl.BlockSpec((TROWS, 128), lambda i: (OROWS // TROWS - 1, 0)),
      out_shape=jax.ShapeDtypeStruct((OROWS, 128), jnp.float32),
      scratch_shapes=[pltpu.VMEM((VTAIL, DIM), jnp.float32)],
      input_output_aliases={1: 0},
  )(lax.slice(wt, (0, VMAIN), (DIM, VOCAB)), main)


CHUNK = 512
NBUF = 5
CHUNKS_TOTAL = NUM_ROWS // CHUNK
CHUNKS_PER_W = CHUNKS_TOTAL // NW
GROUPS = CHUNKS_PER_W // NBUF


def _embed_body(idx_hbm, table_hbm, out_hbm, idx_v, rows_v, gsem, osem):
  wid = lax.axis_index("s") * _NC + lax.axis_index("c")
  row0 = wid * CHUNKS_PER_W

  pltpu.sync_copy(idx_hbm.at[pl.ds(row0, CHUNKS_PER_W)], idx_v)

  def gather(j, b):
    return pltpu.make_async_copy(table_hbm.at[idx_v.at[j]], rows_v.at[b],
                                 gsem.at[b])

  def outcopy(j, b):
    return pltpu.make_async_copy(
        rows_v.at[b], out_hbm.at[pl.ds((row0 + j) * CHUNK, CHUNK)],
        osem.at[b])

  for b in range(NBUF):
    gather(b, b).start()

  def body(g, carry):
    j0 = g * NBUF
    for b in range(NBUF):
      gather(j0 + b, b).wait()
      outcopy(j0 + b, b).start()
    for b in range(NBUF):
      outcopy(j0 + b, b).wait()
      gather(j0 + NBUF + b, b).start()
    return carry

  lax.fori_loop(0, GROUPS - 1, body, 0)

  j0 = (GROUPS - 1) * NBUF
  for b in range(NBUF):
    gather(j0 + b, b).wait()
    outcopy(j0 + b, b).start()
  for b in range(NBUF):
    outcopy(j0 + b, b).wait()


@jax.jit
def _embed(x_flat2d, w_lin):
  mesh = plsc.VectorSubcoreMesh(core_axis_name="c", subcore_axis_name="s")
  run = pl.kernel(
      _embed_body,
      out_type=jax.ShapeDtypeStruct((NUM_ROWS, DIM), jnp.float32),
      mesh=mesh,
      scratch_types=[
          pltpu.VMEM((CHUNKS_PER_W, CHUNK), jnp.int32),
          pltpu.VMEM((NBUF, CHUNK, DIM), jnp.float32),
          pltpu.SemaphoreType.DMA((NBUF,)),
          pltpu.SemaphoreType.DMA((NBUF,)),
      ],
      compiler_params=pltpu.CompilerParams(use_tc_tiling_on_sc=False),
  )
  return run(x_flat2d, w_lin)


SEQ = 200
BATCH = 4096
SROWS = BATCH * DIM // 128


def _tc_format_body(g_ref, out_ref, x_ref):
  c = g_ref[...]
  for j in range(4):
    x_ref[pl.Slice(j, SROWS, 4), :] = c[:, DIM * j:DIM * (j + 1)]
  out_ref[...] = jnp.transpose(x_ref[...], (1, 0))


@jax.jit
def _format(g128):
  return pl.pallas_call(
      _tc_format_body,
      grid=(SEQ,),
      in_specs=[pl.BlockSpec((SROWS, 128), lambda s: (s, 0))],
      out_specs=pl.BlockSpec((DIM, BATCH), lambda s: (s, 0)),
      out_shape=jax.ShapeDtypeStruct((SEQ * DIM, BATCH), jnp.float32),
      scratch_shapes=[pltpu.VMEM((BATCH, DIM), jnp.float32)],
  )(g128)


def kernel(x, weight):
  w128 = _relayout(weight.T)
  w_lin = w128.reshape(VOCAB, DIM)
  x_flat2d = x.T.reshape(CHUNKS_TOTAL, CHUNK).astype(jnp.int32)
  out = _embed(x_flat2d, w_lin)
  out2d = _format(out.reshape(NUM_ROWS * DIM // 128, 128))
  return out2d.reshape(SEQ, DIM, BATCH).transpose(2, 0, 1)
